# Initial kernel scaffold; baseline (speedup 1.0000x reference)
import jax
import jax.numpy as jnp
from jax.experimental import pallas as pl


def kernel(x, edge_index, edge_attr, xt1, xt2, xt3, xt4, xt5, xt6, xt7, xt8, xt9, W1, b1, W2, b2, e1, e2, e3, gamma, beta):
    raise NotImplementedError("write your pallas kernel here")


# speedup vs baseline: 7.7868x; 7.7868x over previous
import functools

import jax
import jax.numpy as jnp
from jax import lax
from jax.experimental import pallas as pl
from jax.experimental.pallas import tpu as pltpu
from jax.experimental.pallas import tpu_sc as plsc

N = 10000
E = 160000
EMB = 300
L = 5
HALF = EMB // 2
HPAD = 160
CW = 32
NTILES = 16
NPAD = 10240
RPT = NPAD // NTILES
CHUNK = 128
SCH = 80
EPAD = 163840
SPMM_CH = EPAD // NTILES // SCH
CNT_CH = EPAD // 2 // NTILES // CHUNK
VOCABS = (119, 8, 12, 15, 10, 6, 7, 3, 3)
TOFF = (0, 119, 127, 139, 154, 164, 170, 177, 180)
TCAT = 192
BN = 1000
NB = N // BN

def _counts_body(oh_hbm, aid_hbm, dst_hbm, init_hbm, out_hbm,
                 acc, aid_v, dst_v, rows_v, sem):
    c = lax.axis_index("c")
    s = lax.axis_index("s")
    pltpu.sync_copy(init_hbm.at[c, pl.ds(s * RPT, RPT)],
                    acc.at[pl.ds(s * RPT, RPT)])
    pltpu.sync_copy(aid_hbm.at[c, s], aid_v)
    pltpu.sync_copy(dst_hbm.at[c, s], dst_v)
    plsc.subcore_barrier()

    def body(j, carry):
        pltpu.async_copy(oh_hbm.at[aid_v.at[j]], rows_v, sem).wait()
        pltpu.sync_copy(rows_v, acc.at[dst_v.at[j]], add=True)
        return carry

    lax.fori_loop(0, CNT_CH, body, 0)
    plsc.subcore_barrier()
    pltpu.sync_copy(acc.at[pl.ds(s * RPT, RPT)],
                    out_hbm.at[c, pl.ds(s * RPT, RPT)])


def _spmm_body(h_hbm, src_hbm, dst_hbm, out_hbm,
               acc, sidx, didx, rows_v, sem0, sem1):
    c = lax.axis_index("c")
    s = lax.axis_index("s")
    pltpu.sync_copy(
        h_hbm.at[pl.ds(c * NPAD + s * RPT, RPT)],
        acc.at[pl.ds(s * RPT, RPT)])

    def load_idx(j, buf):
        pltpu.sync_copy(src_hbm.at[c, s, j], sidx.at[buf])
        pltpu.sync_copy(dst_hbm.at[s, j], didx.at[buf])

    def gather(buf, sem):
        return pltpu.make_async_copy(h_hbm.at[sidx.at[buf]], rows_v.at[buf],
                                     sem)

    load_idx(0, 0)
    plsc.subcore_barrier()
    gather(0, sem0).start()

    def body(j2, carry):
        j = 2 * j2
        load_idx(j + 1, 1)
        gather(1, sem1).start()
        gather(0, sem0).wait()
        pltpu.sync_copy(rows_v.at[0], acc.at[didx.at[0]], add=True)

        @pl.when(j + 2 < SPMM_CH)
        def _():
            load_idx(j + 2, 0)
            gather(0, sem0).start()

        gather(1, sem1).wait()
        pltpu.sync_copy(rows_v.at[1], acc.at[didx.at[1]], add=True)
        return carry

    lax.fori_loop(0, SPMM_CH // 2, body, 0)
    plsc.subcore_barrier()
    pltpu.sync_copy(acc.at[pl.ds(s * RPT, RPT)],
                    out_hbm.at[c, pl.ds(s * RPT, RPT)])


@functools.lru_cache(maxsize=1)
def _sc_kernels():
    mesh = plsc.VectorSubcoreMesh(core_axis_name="c", subcore_axis_name="s")
    params = pltpu.CompilerParams(use_tc_tiling_on_sc=False)
    counts = pl.kernel(
        _counts_body,
        out_type=jax.ShapeDtypeStruct((2, NPAD, CW), jnp.float32),
        mesh=mesh,
        scratch_types=[
            pltpu.VMEM_SHARED((NPAD, CW), jnp.float32),
            pltpu.VMEM((CNT_CH, CHUNK), jnp.int32),
            pltpu.VMEM((CNT_CH, CHUNK), jnp.int32),
            pltpu.VMEM((CHUNK, CW), jnp.float32),
            pltpu.SemaphoreType.DMA,
        ],
        compiler_params=params,
    )
    spmm = pl.kernel(
        _spmm_body,
        out_type=jax.ShapeDtypeStruct((2, NPAD, HPAD), jnp.float32),
        mesh=mesh,
        scratch_types=[
            pltpu.VMEM_SHARED((NPAD, HPAD), jnp.float32),
            pltpu.VMEM((2, SCH), jnp.int32),
            pltpu.VMEM((2, SCH), jnp.int32),
            pltpu.VMEM((2, SCH, HPAD), jnp.float32),
            pltpu.SemaphoreType.DMA,
            pltpu.SemaphoreType.DMA,
        ],
        compiler_params=params,
    )
    return counts, spmm


def _init_body(xt_ref, tcat_ref, out_ref):
    iot = lax.broadcasted_iota(jnp.int32, (BN, TCAT), 1)
    b = jnp.zeros((BN, TCAT), jnp.float32)
    for f in range(9):
        idx = xt_ref[0, f, :] + TOFF[f]
        b = b + (iot == idx[:, None]).astype(jnp.float32)
    h = jnp.dot(b, tcat_ref[...], preferred_element_type=jnp.float32,
                precision=lax.Precision.HIGHEST)
    out_ref[0, :, :HALF] = h[:, :HALF]
    out_ref[1, :, :HALF] = h[:, HALF:]
    out_ref[0, :, HALF:] = jnp.zeros((BN, HPAD - HALF), jnp.float32)
    out_ref[1, :, HALF:] = jnp.zeros((BN, HPAD - HALF), jnp.float32)


def _mlp1_body(neigh_ref, c2_ref, ecat_ref, w1_ref, b1_ref, w2_ref, b2_ref,
               g_ref, stats_ref):
    i = pl.program_id(0)
    aggr = jnp.concatenate([neigh_ref[0, :, :HALF], neigh_ref[1, :, :HALF]],
                           axis=1)
    cc = c2_ref[0] + c2_ref[1]
    aggr = aggr + jnp.dot(cc, ecat_ref[...], preferred_element_type=jnp.float32,
                          precision=lax.Precision.HIGHEST)
    bf = jnp.bfloat16
    hmid = jnp.maximum(
        jnp.dot(aggr.astype(bf), w1_ref[...].astype(bf),
                preferred_element_type=jnp.float32)
        + b1_ref[...], 0.0)
    g = jnp.dot(hmid.astype(bf), w2_ref[...].astype(bf),
                preferred_element_type=jnp.float32) \
        + b2_ref[...]
    g_ref[...] = g
    s = jnp.sum(g, axis=0)

    @pl.when(i == 0)
    def _():
        c = s * (1.0 / BN)
        d = g - c
        stats_ref[0] = s
        stats_ref[1] = jnp.sum(d * d, axis=0)
        stats_ref[2] = c

    @pl.when(i > 0)
    def _():
        d = g - stats_ref[2]
        stats_ref[0] = stats_ref[0] + s
        stats_ref[1] = stats_ref[1] + jnp.sum(d * d, axis=0)


def _bn_body(g_ref, stats_ref, gamma_ref, beta_ref, out_ref, *, relu, split):
    mean = stats_ref[0] * (1.0 / N)
    dm = mean - stats_ref[2]
    var = stats_ref[1] * (1.0 / N) - dm * dm
    scale = gamma_ref[0] / jnp.sqrt(var + 1e-5)
    h = (g_ref[...] - mean) * scale + beta_ref[0]
    if relu:
        h = jnp.maximum(h, 0.0)
    if split:
        out_ref[0, :, :HALF] = h[:, :HALF]
        out_ref[1, :, :HALF] = h[:, HALF:]
        out_ref[0, :, HALF:] = jnp.zeros((BN, HPAD - HALF), jnp.float32)
        out_ref[1, :, HALF:] = jnp.zeros((BN, HPAD - HALF), jnp.float32)
    else:
        out_ref[...] = h


_init_call = pl.pallas_call(
    _init_body,
    grid=(NB,),
    in_specs=[
        pl.BlockSpec((1, 16, BN), lambda i: (i, 0, 0)),
        pl.BlockSpec((TCAT, EMB), lambda i: (0, 0)),
    ],
    out_specs=pl.BlockSpec((2, BN, HPAD), lambda i: (0, i, 0)),
    out_shape=jax.ShapeDtypeStruct((2, NPAD, HPAD), jnp.float32),
)

_mlp1_call = pl.pallas_call(
    _mlp1_body,
    grid=(NB,),
    in_specs=[
        pl.BlockSpec((2, BN, HPAD), lambda i: (0, i, 0)),
        pl.BlockSpec((2, BN, CW), lambda i: (0, i, 0)),
        pl.BlockSpec((CW, EMB), lambda i: (0, 0)),
        pl.BlockSpec((EMB, 2 * EMB), lambda i: (0, 0)),
        pl.BlockSpec((1, 2 * EMB), lambda i: (0, 0)),
        pl.BlockSpec((2 * EMB, EMB), lambda i: (0, 0)),
        pl.BlockSpec((1, EMB), lambda i: (0, 0)),
    ],
    out_specs=[
        pl.BlockSpec((BN, EMB), lambda i: (i, 0)),
        pl.BlockSpec((8, EMB), lambda i: (0, 0)),
    ],
    out_shape=[
        jax.ShapeDtypeStruct((N, EMB), jnp.float32),
        jax.ShapeDtypeStruct((8, EMB), jnp.float32),
    ],
)


def _bn_call(relu, split):
    return pl.pallas_call(
        functools.partial(_bn_body, relu=relu, split=split),
        grid=(NB,),
        in_specs=[
            pl.BlockSpec((BN, EMB), lambda i: (i, 0)),
            pl.BlockSpec((8, EMB), lambda i: (0, 0)),
            pl.BlockSpec((1, EMB), lambda i: (0, 0)),
            pl.BlockSpec((1, EMB), lambda i: (0, 0)),
        ],
        out_specs=(pl.BlockSpec((2, BN, HPAD), lambda i: (0, i, 0))
                   if split else pl.BlockSpec((BN, EMB), lambda i: (i, 0))),
        out_shape=(jax.ShapeDtypeStruct((2, NPAD, HPAD), jnp.float32)
                   if split else jax.ShapeDtypeStruct((N, EMB), jnp.float32)),
    )


_bn_split = _bn_call(relu=True, split=True)
_bn_final = _bn_call(relu=False, split=False)


def kernel(x, edge_index, edge_attr, xt1, xt2, xt3, xt4, xt5, xt6, xt7, xt8,
           xt9, W1, b1, W2, b2, e1, e2, e3, gamma, beta):
    f32 = jnp.float32
    src = edge_index[0].astype(jnp.int32)
    dst = edge_index[1].astype(jnp.int32)
    ea = edge_attr.astype(jnp.int32)

    npad = EPAD - E
    ar = jnp.arange(npad, dtype=jnp.int32)
    src_full = jnp.concatenate([src, ar % N])
    dst_full = jnp.concatenate([dst, N + (ar % 16)])
    src_rs = jnp.stack([src_full, src_full + NPAD]).reshape(2, NTILES,
                                                            SPMM_CH, SCH)
    dst_rs = dst_full.reshape(NTILES, SPMM_CH, SCH)

    aid = ea[:, 0] * 12 + ea[:, 1] * 2 + ea[:, 2]
    aid_full = jnp.concatenate([aid, ar % 276])
    aid_rs = aid_full.reshape(2, NTILES, CNT_CH, CHUNK)
    dst_c = dst_full.reshape(2, NTILES, CNT_CH, CHUNK)

    a = jnp.arange(276)
    oh = jnp.concatenate(
        [(a[:, None] // 12 == jnp.arange(23)[None]).astype(f32),
         ((a[:, None] // 2) % 6 == jnp.arange(6)[None]).astype(f32),
         (a[:, None] % 2 == jnp.arange(2)[None]).astype(f32),
         jnp.zeros((276, 1), f32)], axis=1)
    sl_row = jnp.zeros((CW,), f32).at[22].set(1.0).at[23].set(1.0) \
        .at[29].set(1.0)
    init_c = jnp.stack([jnp.tile(sl_row, (NPAD, 1)),
                        jnp.zeros((NPAD, CW), f32)])

    xt_p = jnp.zeros((16, N), jnp.int32).at[:9].set(x.T.astype(jnp.int32))
    xt_p = xt_p.reshape(16, NB, BN).swapaxes(0, 1)
    tcat = jnp.zeros((TCAT, EMB), f32).at[:183].set(
        jnp.concatenate([xt1, xt2, xt3, xt4, xt5, xt6, xt7, xt8, xt9], axis=0))
    counts_kernel, spmm_kernel = _sc_kernels()
    h_split = _init_call(xt_p, tcat)
    c2 = counts_kernel(oh, aid_rs, dst_c, init_c)

    h = None
    for l in range(L):
        ecat = jnp.zeros((CW, EMB), f32).at[:31].set(
            jnp.concatenate([e1[l], e2[l], e3[l]], axis=0))
        neigh = spmm_kernel(h_split.reshape(2 * NPAD, HPAD), src_rs, dst_rs)
        g, stats = _mlp1_call(neigh, c2, ecat, W1[l], b1[l][None], W2[l],
                              b2[l][None])
        if l != L - 1:
            h_split = _bn_split(g, stats, gamma[l][None], beta[l][None])
        else:
            h = _bn_final(g, stats, gamma[l][None], beta[l][None])
    return h
